# baseline (speedup 1.0000x reference)
import functools

import jax
import jax.numpy as jnp
from jax.experimental import pallas as pl
from jax.experimental.pallas import tpu as pltpu
from jax.experimental.pallas import tpu_sc as plsc
from jax import lax


_BH = 8


def _sc_gather(num_steps, alpha_bars):
    info = plsc.get_sparse_core_info()
    nw = info.num_cores * info.num_subcores
    b = num_steps.shape[0]
    bpw = b // nw
    mesh = plsc.VectorSubcoreMesh(core_axis_name="c", subcore_axis_name="s")

    @functools.partial(
        pl.kernel,
        mesh=mesh,
        out_type=jax.ShapeDtypeStruct((b,), jnp.float32),
        scratch_types=[
            pltpu.VMEM((bpw,), jnp.int32),
            pltpu.VMEM((bpw,), jnp.float32),
            pltpu.SemaphoreType.DMA,
        ],
    )
    def k(idx_hbm, table_hbm, out_hbm, idx_v, vals_v, sem):
        wid = lax.axis_index("s") * info.num_cores + lax.axis_index("c")
        base = wid * bpw
        pltpu.sync_copy(idx_hbm.at[pl.ds(base, bpw)], idx_v)
        pltpu.async_copy(table_hbm.at[idx_v], vals_v, sem).wait()
        pltpu.sync_copy(vals_v, out_hbm.at[pl.ds(base, bpw)])

    return k(num_steps, alpha_bars)


def _tc_body(vals_ref, out_ref, buf_ref, sem_ref):
    vals = vals_ref[...]
    buf_ref[...] = jnp.broadcast_to(vals[None, :, :], buf_ref.shape)

    c, h, w, _ = out_ref.shape
    nj = h // _BH
    copies = []
    for ci in range(c):
        for j in range(nj):
            cp = pltpu.make_async_copy(
                buf_ref,
                out_ref.at[ci, pl.ds(j * _BH, _BH)],
                sem_ref.at[ci * nj + j],
            )
            cp.start()
            copies.append(cp)
    for cp in copies:
        cp.wait()


def kernel(img, num_steps, alpha_bars):
    b, c, h, w = img.shape
    vals = _sc_gather(num_steps, alpha_bars)
    ndma = c * (h // _BH)

    out_t = pl.pallas_call(
        _tc_body,
        in_specs=[pl.BlockSpec(memory_space=pltpu.VMEM)],
        out_specs=pl.BlockSpec(memory_space=pl.ANY),
        out_shape=jax.ShapeDtypeStruct((c, h, w, b), jnp.float32),
        scratch_shapes=[
            pltpu.VMEM((_BH, w, b), jnp.float32),
            pltpu.SemaphoreType.DMA((ndma,)),
        ],
    )(vals.reshape(1, b))
    return jnp.transpose(out_t, (3, 0, 1, 2))


# speedup vs baseline: 1.8782x; 1.8782x over previous
import jax
import jax.numpy as jnp
from jax import lax
from jax.experimental import pallas as pl
from jax.experimental.pallas import tpu as pltpu


_BH = 8
_NSRC = 2


def _body(steps_ref, tab_ref, out_ref, buf_ref, sem_ref):
    steps = steps_ref[...]
    tab = tab_ref[...]
    t = tab.shape[0]
    b = steps.shape[1]
    sub = lax.broadcasted_iota(jnp.int32, (t, b), 0)
    eq = sub == steps
    vals = jnp.sum(jnp.where(eq, tab, 0.0), axis=0, keepdims=True)
    buf_ref[...] = jnp.broadcast_to(vals[None, None, :, :], buf_ref.shape)

    c, h, w, _ = out_ref.shape
    nj = h // _BH
    copies = []
    for ci in range(c):
        for j in range(nj):
            k = ci * nj + j
            cp = pltpu.make_async_copy(
                buf_ref.at[k % _NSRC],
                out_ref.at[ci, pl.ds(j * _BH, _BH)],
                sem_ref.at[k],
            )
            cp.start()
            copies.append(cp)
    for cp in copies:
        cp.wait()


def kernel(img, num_steps, alpha_bars):
    b, c, h, w = img.shape
    t_pad = 1024
    tab_col = jnp.zeros((t_pad, 1), jnp.float32).at[: alpha_bars.shape[0], 0].set(
        alpha_bars
    )
    steps_row = num_steps.reshape(1, b)
    ndma = c * (h // _BH)

    out_t = pl.pallas_call(
        _body,
        in_specs=[
            pl.BlockSpec(memory_space=pltpu.VMEM),
            pl.BlockSpec(memory_space=pltpu.VMEM),
        ],
        out_specs=pl.BlockSpec(memory_space=pl.ANY),
        out_shape=jax.ShapeDtypeStruct((c, h, w, b), jnp.float32),
        scratch_shapes=[
            pltpu.VMEM((_NSRC, _BH, w, b), jnp.float32),
            pltpu.SemaphoreType.DMA((ndma,)),
        ],
    )(steps_row, tab_col)
    return jnp.transpose(out_t, (3, 0, 1, 2))
